# Initial kernel scaffold; baseline (speedup 1.0000x reference)
import jax
import jax.numpy as jnp
from jax import lax
from jax.experimental import pallas as pl
from jax.experimental.pallas import tpu as pltpu
from jax.experimental.pallas import tpu_sc as plsc

B = 16384
F = 26
D = 16
NC, NS, L = 2, 16, 16
NW = NC * NS
RW = B // NW
CH = 64
NCH = RW // CH
G = CH // L
IPC = F * CH
KB = IPC // 128


def _fm_body(idx_hbm, val_hbm, emb_hbm, w_hbm, b_hbm, out_hbm,
             idx_v, val_v, rows_v, wv_v, out_v, bv_v, sem_e, sem_w):
    wid = lax.axis_index("s") * NC + lax.axis_index("c")
    pltpu.sync_copy(b_hbm, bv_v)
    iota = lax.iota(jnp.int32, L)

    def chunk_body(c, carry):
        pltpu.sync_copy(idx_hbm.at[wid, c], idx_v)
        pltpu.sync_copy(val_hbm.at[wid, c], val_v)
        cps = []
        for k in range(KB):
            cps.append(pltpu.async_copy(
                emb_hbm.at[idx_v.at[k]], rows_v.at[pl.ds(k * 128, 128)],
                sem_e))
        for k in range(KB):
            cps.append(pltpu.async_copy(
                w_hbm.at[idx_v.at[k]], wv_v.at[pl.ds(k * 128, 128)], sem_w))
        for cp in cps:
            cp.wait()
        bvec = bv_v[...]
        for g in range(G):
            tv = [val_v[pl.ds((g * F + j) * L, L)] for j in range(F)]
            lin = bvec
            for j in range(F):
                lin = lin + tv[j] * wv_v[pl.ds((g * F + j) * L, L)]

            def d_body(d, acc, g=g, tv=tv):
                dvec = jnp.full((L,), d, dtype=jnp.int32)
                s = jnp.zeros((L,), jnp.float32)
                for j in range(F):
                    e = plsc.load_gather(
                        rows_v, [iota + (g * F + j) * L, dvec])
                    t = tv[j] * e
                    s = s + t
                    acc = acc - t * t
                return acc + s * s

            acc = lax.fori_loop(0, D, d_body, jnp.zeros((L,), jnp.float32))
            out_v[pl.ds(c * CH + g * L, L)] = lin + 0.5 * acc
        return carry

    lax.fori_loop(0, NCH, chunk_body, 0)
    pltpu.sync_copy(out_v, out_hbm.at[pl.ds(wid * RW, RW)])


_MESH = plsc.VectorSubcoreMesh(
    core_axis_name="c", subcore_axis_name="s",
    num_cores=NC, num_subcores=NS)

_FM = pl.kernel(
    _fm_body,
    out_type=jax.ShapeDtypeStruct((B,), jnp.float32),
    mesh=_MESH,
    scratch_types=[
        pltpu.VMEM((KB, 128), jnp.int32),
        pltpu.VMEM((IPC,), jnp.float32),
        pltpu.VMEM((IPC, D), jnp.float32),
        pltpu.VMEM((IPC,), jnp.float32),
        pltpu.VMEM((RW,), jnp.float32),
        pltpu.VMEM((L,), jnp.float32),
        pltpu.SemaphoreType.DMA,
        pltpu.SemaphoreType.DMA,
    ],
)


def kernel(feat_idx, feat_val, feature_embed, linear_w, linear_b):
    idx = feat_idx.astype(jnp.int32).reshape(NW, NCH, G, L, F)
    idx = idx.transpose(0, 1, 2, 4, 3).reshape(NW, NCH, KB, 128)
    val = feat_val.astype(jnp.float32).reshape(NW, NCH, G, L, F)
    val = val.transpose(0, 1, 2, 4, 3).reshape(NW, NCH, IPC)
    bvec = jnp.broadcast_to(linear_b.astype(jnp.float32), (L,))
    return _FM(idx, val, feature_embed, linear_w, bvec)


# speedup vs baseline: 1.2195x; 1.2195x over previous
import jax
import jax.numpy as jnp
from jax import lax
from jax.experimental import pallas as pl
from jax.experimental.pallas import tpu as pltpu
from jax.experimental.pallas import tpu_sc as plsc

B = 16384
F = 26
D = 16
NC, NS, L = 2, 16, 16
NW = NC * NS
RW = B // NW
CH = 64
NCH = RW // CH
G = CH // L
IPC = F * CH
KB = IPC // 128


def _fm_body(idx_hbm, val_hbm, emb_hbm, w_hbm, b_hbm, out_hbm,
             idx_v, val_v, rows_v, wv_v, out_v, bv_v, sem_e, sem_w):
    wid = lax.axis_index("s") * NC + lax.axis_index("c")
    pltpu.sync_copy(b_hbm, bv_v)
    iota = lax.iota(jnp.int32, L)

    def chunk_body(c, carry):
        pltpu.sync_copy(idx_hbm.at[wid, c], idx_v)
        pltpu.sync_copy(val_hbm.at[wid, c], val_v)
        cps = []
        for k in range(KB):
            cps.append(pltpu.async_copy(
                emb_hbm.at[idx_v.at[k]], rows_v.at[pl.ds(k * 128, 128)],
                sem_e))
        for k in range(KB):
            cps.append(pltpu.async_copy(
                w_hbm.at[idx_v.at[k]], wv_v.at[pl.ds(k * 128, 128)], sem_w))
        for cp in cps:
            cp.wait()
        bvec = bv_v[...]
        for g in range(G):
            tv = [val_v[pl.ds((g * F + j) * L, L)] for j in range(F)]
            lin = bvec
            for j in range(F):
                lin = lin + tv[j] * wv_v[pl.ds((g * F + j) * L, L)]

            def d_body(d, acc, g=g, tv=tv):
                dvec = jnp.full((L,), d, dtype=jnp.int32)
                s = jnp.zeros((L,), jnp.float32)
                for j in range(F):
                    e = plsc.load_gather(
                        rows_v, [iota + (g * F + j) * L, dvec])
                    t = tv[j] * e
                    s = s + t
                    acc = acc - t * t
                return acc + s * s

            acc = lax.fori_loop(0, D, d_body, jnp.zeros((L,), jnp.float32))
            out_v[pl.ds(c * CH + g * L, L)] = lin + 0.5 * acc
        return carry

    lax.fori_loop(0, NCH, chunk_body, 0)
    pltpu.sync_copy(out_v, out_hbm.at[pl.ds(wid * RW, RW)])


_MESH = plsc.VectorSubcoreMesh(
    core_axis_name="c", subcore_axis_name="s",
    num_cores=NC, num_subcores=NS)

_FM = pl.kernel(
    _fm_body,
    out_type=jax.ShapeDtypeStruct((B,), jnp.float32),
    mesh=_MESH,
    compiler_params=pltpu.CompilerParams(
        needs_layout_passes=False, use_tc_tiling_on_sc=False),
    scratch_types=[
        pltpu.VMEM((KB, 128), jnp.int32),
        pltpu.VMEM((IPC,), jnp.float32),
        pltpu.VMEM((IPC, D), jnp.float32),
        pltpu.VMEM((IPC,), jnp.float32),
        pltpu.VMEM((RW,), jnp.float32),
        pltpu.VMEM((L,), jnp.float32),
        pltpu.SemaphoreType.DMA,
        pltpu.SemaphoreType.DMA,
    ],
)


def kernel(feat_idx, feat_val, feature_embed, linear_w, linear_b):
    idx = feat_idx.astype(jnp.int32).reshape(NW, NCH, G, L, F)
    idx = idx.transpose(0, 1, 2, 4, 3).reshape(NW, NCH, KB, 128)
    val = feat_val.astype(jnp.float32).reshape(NW, NCH, G, L, F)
    val = val.transpose(0, 1, 2, 4, 3).reshape(NW, NCH, IPC)
    bvec = jnp.broadcast_to(linear_b.astype(jnp.float32), (L,))
    return _FM(idx, val, feature_embed, linear_w, bvec)
